# baseline (speedup 1.0000x reference)
import functools

import jax
import jax.numpy as jnp
from jax import lax
from jax.experimental import pallas as pl
from jax.experimental.pallas import tpu as pltpu
from jax.experimental.pallas import tpu_sc as plsc

B, S, D = 4, 8192, 768
G, EPG = 4, 16
E = G * EPG
H = 64
T = B * S
WPAD = 192
LCOLS = 128
BT = 512

NC, NS = 2, 16
NW = NC * NS
TPW = T // NW
CH = 128
NCHUNK = TPW // CH


def _tc_logits_body(xref, wref, biasref, w2ref, b2ref, outref):
    xb = xref[...]
    y = lax.dot_general(xb, wref[...], (((1,), (1,)), ((), ())),
                        preferred_element_type=jnp.float32)
    r = jnp.maximum(y + biasref[...], 0.0)
    cpre = jnp.sum(r * w2ref[...], axis=1) + b2ref[0, 0]
    c = jax.nn.sigmoid(cpre)
    outref[...] = jnp.concatenate(
        [y[:, :E], y[:, 128:132], c[:, None],
         jnp.zeros((BT, LCOLS - E - G - 1), jnp.float32)], axis=1)


def _tc_logits(xf, wall, bias, w2pad, b2s):
    return pl.pallas_call(
        _tc_logits_body,
        grid=(T // BT,),
        in_specs=[
            pl.BlockSpec((BT, D), lambda i: (i, 0)),
            pl.BlockSpec((WPAD, D), lambda i: (0, 0)),
            pl.BlockSpec((1, WPAD), lambda i: (0, 0)),
            pl.BlockSpec((1, WPAD), lambda i: (0, 0)),
            pl.BlockSpec(memory_space=pltpu.SMEM),
        ],
        out_specs=pl.BlockSpec((BT, LCOLS), lambda i: (i, 0)),
        out_shape=jax.ShapeDtypeStruct((T, LCOLS), jnp.float32),
    )(xf, wall, bias, w2pad, b2s)


def _sc_route_body(lg_hbm, disp_hbm, comb_hbm, rp_hbm, part_hbm,
                   inb, db, cb, rb, pb):
    wid = lax.axis_index("c") * NS + lax.axis_index("s")
    iota = lax.iota(jnp.int32, 16)
    neg = jnp.float32(-3.4e38)

    acc0 = tuple(jnp.zeros((16,), jnp.float32) for _ in range(8))
    acc = acc0
    for ch in range(NCHUNK):
        base = wid * TPW + ch * CH
        pltpu.sync_copy(lg_hbm.at[pl.ds(base, CH)], inb)

        def body(i, acc):
            e0 = inb[i, 0:16]
            e1 = inb[i, 16:32]
            e2 = inb[i, 32:48]
            e3 = inb[i, 48:64]
            gv = inb[i, 64:80]
            c = gv[4]
            gmask = iota < 4
            glm = jnp.max(jnp.where(gmask, gv, neg))
            gidx = jnp.min(jnp.where((gv == glm) & gmask, iota, 16))
            gsum = jnp.sum(jnp.where(gmask, jnp.exp(gv - glm), 0.0))
            ce = jnp.where(gidx == 0, e0,
                           jnp.where(gidx == 1, e1,
                                     jnp.where(gidx == 2, e2, e3)))
            ex = jnp.exp(ce - jnp.max(ce))
            es = jnp.sum(ex)
            p = ex / jnp.full((16,), es)
            m1 = jnp.max(p)
            i1 = jnp.min(jnp.where(p == m1, iota, 16))
            pm = jnp.where(iota == i1, -1.0, p)
            m2 = jnp.max(pm)
            i2 = jnp.min(jnp.where(pm == m2, iota, 16))
            sel2 = (c * 2.0 >= 2.0)
            m2k = jnp.where(sel2, m2, jnp.float32(0.0))
            den = jnp.maximum(m1 + m2k, 1e-20)
            oh1 = iota == i1
            oh2 = iota == i2
            one = jnp.float32(1.0)
            zero = jnp.float32(0.0)
            disp = jnp.where(oh1, one, zero) + jnp.where(oh2 & sel2, one, zero)
            comb = (jnp.where(oh1, m1, zero)
                    + jnp.where(oh2, m2k, zero)) / jnp.full((16,), den)
            basev = p / jnp.full((16,), gsum)
            zz = jnp.zeros((16,), jnp.float32)
            accl = list(acc)
            for g in range(G):
                hit = gidx == g
                dv = jnp.where(hit, disp, zz)
                bv = jnp.where(hit, basev, zz)
                db[i, g * 16:(g + 1) * 16] = dv
                cb[i, g * 16:(g + 1) * 16] = jnp.where(hit, comb, zz)
                rb[i, g * 16:(g + 1) * 16] = bv
                accl[g] = accl[g] + bv
                accl[4 + g] = accl[4 + g] + dv
            return tuple(accl)

        acc = lax.fori_loop(0, CH, body, acc)
        pltpu.sync_copy(db, disp_hbm.at[pl.ds(base, CH)])
        pltpu.sync_copy(cb, comb_hbm.at[pl.ds(base, CH)])
        pltpu.sync_copy(rb, rp_hbm.at[pl.ds(base, CH)])

    for g in range(8):
        pb[g * 16:(g + 1) * 16] = acc[g]
    pltpu.sync_copy(pb, part_hbm.at[wid])


def _sc_route(logits):
    mesh = plsc.VectorSubcoreMesh(core_axis_name="c", subcore_axis_name="s")
    out_type = (
        jax.ShapeDtypeStruct((T, E), jnp.float32),
        jax.ShapeDtypeStruct((T, E), jnp.float32),
        jax.ShapeDtypeStruct((T, E), jnp.float32),
        jax.ShapeDtypeStruct((NW, 128), jnp.float32),
    )
    scratch = [
        pltpu.VMEM((CH, LCOLS), jnp.float32),
        pltpu.VMEM((CH, E), jnp.float32),
        pltpu.VMEM((CH, E), jnp.float32),
        pltpu.VMEM((CH, E), jnp.float32),
        pltpu.VMEM((128,), jnp.float32),
    ]
    fn = functools.partial(
        pl.kernel, out_type=out_type, mesh=mesh, scratch_types=scratch,
        compiler_params=pltpu.CompilerParams(needs_layout_passes=False),
    )(_sc_route_body)
    return fn(logits)


def _tc_aux_body(pref, outref):
    p = pref[...]
    rppe = jnp.sum(p[:, :E], axis=0) / T
    usage = jnp.sum(p[:, E:], axis=0) / T
    outref[0, 0] = jnp.sum(rppe * usage) * E


def _tc_aux(partials):
    return pl.pallas_call(
        _tc_aux_body,
        in_specs=[pl.BlockSpec((NW, 128), lambda: (0, 0))],
        out_specs=pl.BlockSpec(memory_space=pltpu.SMEM),
        out_shape=jax.ShapeDtypeStruct((1, 1), jnp.float32),
    )(partials)


def kernel(x, Wg, We, W1, b1, W2, b2):
    xf = x.reshape(T, D)
    wall = jnp.concatenate(
        [We.reshape(E, D), W1, Wg, jnp.zeros((WPAD - E - H - G, D), jnp.float32)],
        axis=0)
    bias = jnp.zeros((1, WPAD), jnp.float32).at[0, E:E + H].set(b1)
    w2pad = jnp.zeros((1, WPAD), jnp.float32).at[0, E:E + H].set(W2[0])
    b2s = b2.reshape(1, 1)

    logits = _tc_logits(xf, wall, bias, w2pad, b2s)
    disp, comb, rp, partials = _sc_route(logits)
    aux = _tc_aux(partials)[0, 0]
    return (disp.reshape(B, S, E), comb.reshape(B, S, E),
            rp.reshape(B, S, E), aux)
